# Initial kernel scaffold; baseline (speedup 1.0000x reference)
import jax
import jax.numpy as jnp
from jax.experimental import pallas as pl


def kernel(x, conv_w, bn_gamma, bn_beta, bn_mean, bn_var):
    raise NotImplementedError("write your pallas kernel here")


# speedup vs baseline: 1.0451x; 1.0451x over previous
import jax
import jax.numpy as jnp
from jax.experimental import pallas as pl
from jax.experimental.pallas import tpu as pltpu

_MIB = 1024 * 1024


def _fused_kernel(x_ref, w_ref, a_ref, b_ref, o_ref):
    s = jnp.sum(x_ref[0], axis=1, keepdims=True)
    y = jax.lax.dot_general(w_ref[...], s,
                            (((1,), (0,)), ((), ())),
                            preferred_element_type=jnp.float32)
    z = jnp.maximum(y * a_ref[...] + b_ref[...], 0.0)
    o_ref[0] = jnp.broadcast_to(z, o_ref.shape[1:]).astype(o_ref.dtype)


def kernel(x, conv_w, bn_gamma, bn_beta, bn_mean, bn_var, eps=1e-5):
    N, Cin, H, W = x.shape
    Cout = conv_w.shape[0]
    HW = H * W

    scale = (bn_gamma.astype(jnp.float32)
             / jnp.sqrt(bn_var.astype(jnp.float32) + eps))
    bias = bn_beta.astype(jnp.float32) - bn_mean.astype(jnp.float32) * scale
    alpha = (scale * (1.0 / HW))[:, None]
    beta = bias[:, None]
    wr = conv_w.reshape(Cout, Cin).astype(jnp.float32)

    x3 = x.reshape(N, Cin, HW)
    itemsize = jnp.dtype(x.dtype).itemsize

    out = pl.pallas_call(
        _fused_kernel,
        out_shape=jax.ShapeDtypeStruct((N, Cout, HW), x.dtype),
        grid=(N,),
        in_specs=[
            pl.BlockSpec((1, Cin, HW), lambda n: (n, 0, 0)),
            pl.BlockSpec((Cout, Cin), lambda n: (0, 0)),
            pl.BlockSpec((Cout, 1), lambda n: (0, 0)),
            pl.BlockSpec((Cout, 1), lambda n: (0, 0)),
        ],
        out_specs=pl.BlockSpec((1, Cout, HW), lambda n: (n, 0, 0)),
        compiler_params=pltpu.CompilerParams(
            dimension_semantics=("parallel",),
            vmem_limit_bytes=48 * _MIB),
        cost_estimate=pl.CostEstimate(
            flops=int(N * Cin * HW + 2 * N * Cin * Cout),
            transcendentals=0,
            bytes_accessed=int(N * Cin * HW * itemsize
                               + N * Cout * HW * itemsize
                               + Cin * Cout * 4)),
    )(x3, wr, alpha, beta)

    return out.reshape(N, Cout, H, W)
